# Initial kernel scaffold; baseline (speedup 1.0000x reference)
import jax
import jax.numpy as jnp
from jax.experimental import pallas as pl


def kernel(hidden_state, embedding_weight):
    raise NotImplementedError("write your pallas kernel here")


# speedup vs baseline: 1.1267x; 1.1267x over previous
import functools

import jax
import jax.numpy as jnp
from jax.experimental import pallas as pl
from jax.experimental.pallas import tpu as pltpu

_CODEBOOK = 8192
_EMBED = 256
_N_CHUNK = 1024
_GROUPS = ((0, 2736), (2736, 5472), (5472, 8192))


def _vq_body(hs_ref, hn_ref, e_ref, out_ref):
    hs = hs_ref[0]
    hn = hn_ref[0]
    hs_b = hs.astype(jnp.bfloat16)
    accs = [None, None, None]
    for c in range(_CODEBOOK // _N_CHUNK):
        t0 = c * _N_CHUNK
        t1 = t0 + _N_CHUNK
        ec = e_ref[pl.ds(t0, _N_CHUNK), :]
        scores = jax.lax.dot_general(
            ec.astype(jnp.bfloat16), hs_b, (((1,), (0,)), ((), ())),
            preferred_element_type=jnp.float32,
        )
        e_norm = jnp.sum(ec * ec, axis=1, keepdims=True)
        dist = (hn + e_norm) - 2.0 * scores
        iota = jax.lax.broadcasted_iota(jnp.int32, dist.shape, 0)
        for g, (lo, hi) in enumerate(_GROUPS):
            o0, o1 = max(lo, t0), min(hi, t1)
            if o0 >= o1:
                continue
            if o0 == t0 and o1 == t1:
                dg = dist
            else:
                in_g = (iota >= (o0 - t0)) & (iota < (o1 - t0))
                dg = jnp.where(in_g, dist, jnp.inf)
            cmin = jnp.min(dg, axis=0, keepdims=True)
            cidx = jnp.min(
                jnp.where(dg == cmin, iota + t0, _CODEBOOK),
                axis=0, keepdims=True,
            )
            if accs[g] is None:
                accs[g] = (cmin, cidx)
            else:
                rmin, ridx = accs[g]
                better = cmin < rmin
                accs[g] = (jnp.minimum(rmin, cmin),
                           jnp.where(better, cidx, ridx))
    (m0, i0), (m1, i1), (m2, i2) = accs
    b0 = m0.astype(jnp.bfloat16).astype(jnp.float32)
    keep0 = b0 <= m1
    m01 = jnp.where(keep0, b0, m1)
    i01 = jnp.where(keep0, i0, i1)
    b01 = m01.astype(jnp.bfloat16).astype(jnp.float32)
    keep01 = b01 <= m2
    out_ref[0] = jnp.where(keep01, i01, i2)


@functools.partial(jax.jit, static_argnames=())
def kernel(hidden_state, embedding_weight):
    b, t, ch, h, w = hidden_state.shape
    hs3 = hidden_state.reshape(b * t, ch, h * w)
    hs2d = jnp.transpose(hidden_state, (0, 1, 3, 4, 2)).reshape(-1, ch)
    hn = jnp.sum(hs2d ** 2, axis=1).reshape(b * t, 1, h * w)
    n_slices = b * t
    out = pl.pallas_call(
        _vq_body,
        grid=(n_slices,),
        in_specs=[
            pl.BlockSpec((1, ch, h * w), lambda i: (i, 0, 0)),
            pl.BlockSpec((1, 1, h * w), lambda i: (i, 0, 0)),
            pl.BlockSpec((_CODEBOOK, _EMBED), lambda i: (0, 0)),
        ],
        out_specs=pl.BlockSpec((1, 1, h * w), lambda i: (i, 0, 0)),
        out_shape=jax.ShapeDtypeStruct((n_slices, 1, h * w), jnp.int32),
        compiler_params=pltpu.CompilerParams(
            dimension_semantics=("parallel",),
        ),
    )(hs3, hn, embedding_weight)
    return out.reshape(b, t, h, w)


# speedup vs baseline: 1.2616x; 1.1198x over previous
import functools

import jax
import jax.numpy as jnp
from jax.experimental import pallas as pl
from jax.experimental.pallas import tpu as pltpu

_CODEBOOK = 8192
_EMBED = 256
_GROUPS = ((0, 2736), (2736, 5472), (5472, 8192))


def _vq_body(hs_ref, hn_ref, e_ref, en_ref, out_ref):
    hs = hs_ref[0]
    hn = hn_ref[0]
    accs = []
    for lo, hi in _GROUPS:
        ec = e_ref[pl.ds(lo, hi - lo), :]
        scores2 = jax.lax.dot_general(
            ec, hs, (((1,), (0,)), ((), ())),
            preferred_element_type=jnp.float32,
        )
        en = en_ref[pl.ds(lo, hi - lo), :]
        dist = (hn + en) + scores2
        cmin = jnp.min(dist, axis=0, keepdims=True)
        iota = jax.lax.broadcasted_iota(jnp.int32, dist.shape, 0)
        cidx = jnp.min(
            jnp.where(dist == cmin, iota + lo, _CODEBOOK),
            axis=0, keepdims=True,
        )
        accs.append((cmin, cidx))
    (m0, i0), (m1, i1), (m2, i2) = accs
    b0 = m0.astype(jnp.bfloat16).astype(jnp.float32)
    keep0 = b0 <= m1
    m01 = jnp.where(keep0, b0, m1)
    i01 = jnp.where(keep0, i0, i1)
    b01 = m01.astype(jnp.bfloat16).astype(jnp.float32)
    keep01 = b01 <= m2
    out_ref[0] = jnp.where(keep01, i01, i2)


@functools.partial(jax.jit, static_argnames=())
def kernel(hidden_state, embedding_weight):
    b, t, ch, h, w = hidden_state.shape
    hs3 = hidden_state.reshape(b * t, ch, h * w)
    hs3_bf = (-2.0 * hs3).astype(jnp.bfloat16)
    hs2d = jnp.transpose(hidden_state, (0, 1, 3, 4, 2)).reshape(-1, ch)
    hn = jnp.sum(hs2d ** 2, axis=1).reshape(b * t, 1, h * w)
    e_bf = embedding_weight.astype(jnp.bfloat16)
    en = jnp.sum(embedding_weight ** 2, axis=1).reshape(_CODEBOOK, 1)
    n_slices = b * t
    out = pl.pallas_call(
        _vq_body,
        grid=(n_slices,),
        in_specs=[
            pl.BlockSpec((1, ch, h * w), lambda i: (i, 0, 0)),
            pl.BlockSpec((1, 1, h * w), lambda i: (i, 0, 0)),
            pl.BlockSpec((_CODEBOOK, _EMBED), lambda i: (0, 0)),
            pl.BlockSpec((_CODEBOOK, 1), lambda i: (0, 0)),
        ],
        out_specs=pl.BlockSpec((1, 1, h * w), lambda i: (i, 0, 0)),
        out_shape=jax.ShapeDtypeStruct((n_slices, 1, h * w), jnp.int32),
        compiler_params=pltpu.CompilerParams(
            dimension_semantics=("parallel",),
        ),
    )(hs3_bf, hn, e_bf, en)
    return out.reshape(b, t, h, w)


# speedup vs baseline: 1.7628x; 1.3972x over previous
import functools

import jax
import jax.numpy as jnp
from jax.experimental import pallas as pl
from jax.experimental.pallas import tpu as pltpu

_CODEBOOK = 8192
_EMBED = 256
_GROUPS = ((0, 2736), (2736, 5472), (5472, 8192))


def _vq_body(hs_ref, hn_ref, e_ref, en_ref, out_ref):
    hs = hs_ref[0]
    hn = hn_ref[0]
    accs = []
    for lo, hi in _GROUPS:
        ec = e_ref[pl.ds(lo, hi - lo), :]
        scores2 = jax.lax.dot_general(
            ec, hs, (((1,), (0,)), ((), ())),
            preferred_element_type=jnp.float32,
        )
        en = en_ref[pl.ds(lo, hi - lo), :]
        dist = (hn + en) + scores2
        cmin = jnp.min(dist, axis=0, keepdims=True)
        cidx = (jnp.argmin(dist, axis=0).astype(jnp.int32) + lo)[None, :]
        accs.append((cmin, cidx))
    (m0, i0), (m1, i1), (m2, i2) = accs
    b0 = m0.astype(jnp.bfloat16).astype(jnp.float32)
    keep0 = b0 <= m1
    m01 = jnp.where(keep0, b0, m1)
    i01 = jnp.where(keep0, i0, i1)
    b01 = m01.astype(jnp.bfloat16).astype(jnp.float32)
    keep01 = b01 <= m2
    out_ref[0] = jnp.where(keep01, i01, i2)


@functools.partial(jax.jit, static_argnames=())
def kernel(hidden_state, embedding_weight):
    b, t, ch, h, w = hidden_state.shape
    hs3 = hidden_state.reshape(b * t, ch, h * w)
    hs3_bf = (-2.0 * hs3).astype(jnp.bfloat16)
    hs2d = jnp.transpose(hidden_state, (0, 1, 3, 4, 2)).reshape(-1, ch)
    hn = jnp.sum(hs2d ** 2, axis=1).reshape(b * t, 1, h * w)
    e_bf = embedding_weight.astype(jnp.bfloat16)
    en = jnp.sum(embedding_weight ** 2, axis=1).reshape(_CODEBOOK, 1)
    n_slices = b * t
    out = pl.pallas_call(
        _vq_body,
        grid=(n_slices,),
        in_specs=[
            pl.BlockSpec((1, ch, h * w), lambda i: (i, 0, 0)),
            pl.BlockSpec((1, 1, h * w), lambda i: (i, 0, 0)),
            pl.BlockSpec((_CODEBOOK, _EMBED), lambda i: (0, 0)),
            pl.BlockSpec((_CODEBOOK, 1), lambda i: (0, 0)),
        ],
        out_specs=pl.BlockSpec((1, 1, h * w), lambda i: (i, 0, 0)),
        out_shape=jax.ShapeDtypeStruct((n_slices, 1, h * w), jnp.int32),
        compiler_params=pltpu.CompilerParams(
            dimension_semantics=("parallel",),
        ),
    )(hs3_bf, hn, e_bf, en)
    return out.reshape(b, t, h, w)


# speedup vs baseline: 1.9916x; 1.1298x over previous
import functools

import jax
import jax.numpy as jnp
from jax.experimental import pallas as pl
from jax.experimental.pallas import tpu as pltpu

_CODEBOOK = 8192
_EMBED = 256
_GROUPS = ((0, 2736), (2736, 5472), (5472, 8192))


def _vq_body(hs_ref, hn_ref, e_ref, en_ref, out_ref):
    hs = hs_ref[0]
    hn = hn_ref[0]
    ntok = hn.shape[-1]
    _R = 16
    pos = jax.lax.broadcasted_iota(jnp.int32, (_R, ntok), 0)
    accs = []
    for lo, hi in _GROUPS:
        n_g = hi - lo
        ec = e_ref[pl.ds(lo, n_g), :]
        scores2 = jax.lax.dot_general(
            ec, hs, (((1,), (0,)), ((), ())),
            preferred_element_type=jnp.float32,
        )
        en = en_ref[pl.ds(lo, n_g), :]
        av = None
        ai = None
        for c in range(n_g // _R):
            d = (hn + en[c * _R:(c + 1) * _R, :]) + scores2[c * _R:(c + 1) * _R, :]
            if av is None:
                av, ai = d, jnp.zeros((_R, ntok), jnp.int32)
            else:
                pred = d < av
                av = jnp.minimum(av, d)
                ai = jnp.where(pred, c, ai)
        aidx = ai * _R + pos + lo
        cmin = jnp.min(av, axis=0, keepdims=True)
        cidx = jnp.min(
            jnp.where(av == cmin, aidx, _CODEBOOK), axis=0, keepdims=True
        )
        accs.append((cmin, cidx))
    (m0, i0), (m1, i1), (m2, i2) = accs
    b0 = m0.astype(jnp.bfloat16).astype(jnp.float32)
    keep0 = b0 <= m1
    m01 = jnp.where(keep0, b0, m1)
    i01 = jnp.where(keep0, i0, i1)
    b01 = m01.astype(jnp.bfloat16).astype(jnp.float32)
    keep01 = b01 <= m2
    out_ref[0] = jnp.where(keep01, i01, i2)


@functools.partial(jax.jit, static_argnames=())
def kernel(hidden_state, embedding_weight):
    b, t, ch, h, w = hidden_state.shape
    hs3 = hidden_state.reshape(b * t, ch, h * w)
    hs3_bf = (-2.0 * hs3).astype(jnp.bfloat16)
    hs2d = jnp.transpose(hidden_state, (0, 1, 3, 4, 2)).reshape(-1, ch)
    hn = jnp.sum(hs2d ** 2, axis=1).reshape(b * t, 1, h * w)
    e_bf = embedding_weight.astype(jnp.bfloat16)
    en = jnp.sum(embedding_weight ** 2, axis=1).reshape(_CODEBOOK, 1)
    n_slices = b * t
    out = pl.pallas_call(
        _vq_body,
        grid=(n_slices,),
        in_specs=[
            pl.BlockSpec((1, ch, h * w), lambda i: (i, 0, 0)),
            pl.BlockSpec((1, 1, h * w), lambda i: (i, 0, 0)),
            pl.BlockSpec((_CODEBOOK, _EMBED), lambda i: (0, 0)),
            pl.BlockSpec((_CODEBOOK, 1), lambda i: (0, 0)),
        ],
        out_specs=pl.BlockSpec((1, 1, h * w), lambda i: (i, 0, 0)),
        out_shape=jax.ShapeDtypeStruct((n_slices, 1, h * w), jnp.int32),
        compiler_params=pltpu.CompilerParams(
            dimension_semantics=("parallel",),
        ),
    )(hs3_bf, hn, e_bf, en)
    return out.reshape(b, t, h, w)


# speedup vs baseline: 2.0701x; 1.0394x over previous
import functools

import jax
import jax.numpy as jnp
from jax.experimental import pallas as pl
from jax.experimental.pallas import tpu as pltpu

_CODEBOOK = 8192
_EMBED = 256
_GROUPS = ((0, 2736), (2736, 5472), (5472, 8192))


def _vq_body(hs_ref, e_ref, en_ref, out_ref):
    hs_f = hs_ref[0]
    hn = jnp.sum(hs_f * hs_f, axis=0, keepdims=True)
    hs = (hs_f * -2.0).astype(jnp.bfloat16)
    ntok = hn.shape[-1]
    _R = 16
    pos = jax.lax.broadcasted_iota(jnp.int32, (_R, ntok), 0)
    accs = []
    for lo, hi in _GROUPS:
        n_g = hi - lo
        ec = e_ref[pl.ds(lo, n_g), :]
        scores2 = jax.lax.dot_general(
            ec, hs, (((1,), (0,)), ((), ())),
            preferred_element_type=jnp.float32,
        )
        en = en_ref[pl.ds(lo, n_g), :]
        av = None
        ai = None
        for c in range(n_g // _R):
            d = (hn + en[c * _R:(c + 1) * _R, :]) + scores2[c * _R:(c + 1) * _R, :]
            if av is None:
                av, ai = d, jnp.zeros((_R, ntok), jnp.int32)
            else:
                pred = d < av
                av = jnp.minimum(av, d)
                ai = jnp.where(pred, c, ai)
        aidx = ai * _R + pos + lo
        cmin = jnp.min(av, axis=0, keepdims=True)
        cidx = jnp.min(
            jnp.where(av == cmin, aidx, _CODEBOOK), axis=0, keepdims=True
        )
        accs.append((cmin, cidx))
    (m0, i0), (m1, i1), (m2, i2) = accs
    b0 = m0.astype(jnp.bfloat16).astype(jnp.float32)
    keep0 = b0 <= m1
    m01 = jnp.where(keep0, b0, m1)
    i01 = jnp.where(keep0, i0, i1)
    b01 = m01.astype(jnp.bfloat16).astype(jnp.float32)
    keep01 = b01 <= m2
    out_ref[0] = jnp.where(keep01, i01, i2)


@functools.partial(jax.jit, static_argnames=())
def kernel(hidden_state, embedding_weight):
    b, t, ch, h, w = hidden_state.shape
    hs3 = hidden_state.reshape(b * t, ch, h * w)
    e_bf = embedding_weight.astype(jnp.bfloat16)
    en = jnp.sum(embedding_weight ** 2, axis=1).reshape(_CODEBOOK, 1)
    n_slices = b * t
    out = pl.pallas_call(
        _vq_body,
        grid=(n_slices,),
        in_specs=[
            pl.BlockSpec((1, ch, h * w), lambda i: (i, 0, 0)),
            pl.BlockSpec((_CODEBOOK, _EMBED), lambda i: (0, 0)),
            pl.BlockSpec((_CODEBOOK, 1), lambda i: (0, 0)),
        ],
        out_specs=pl.BlockSpec((1, 1, h * w), lambda i: (i, 0, 0)),
        out_shape=jax.ShapeDtypeStruct((n_slices, 1, h * w), jnp.int32),
        compiler_params=pltpu.CompilerParams(
            dimension_semantics=("parallel",),
        ),
    )(hs3, e_bf, en)
    return out.reshape(b, t, h, w)
